# baseline (speedup 1.0000x reference)
import functools
import jax
import jax.numpy as jnp
from jax import lax
from jax.experimental import pallas as pl
from jax.experimental.pallas import tpu as pltpu
from jax.experimental.pallas import tpu_sc as plsc

VOCAB = 100000
EMBED = 128
BATCH = 16384
NSAMP = 64

NC = 2
NS = 16
NW = NC * NS

CHUNK = 128
SMALL_CH = BATCH // NW // CHUNK
NOISE_CH = BATCH * NSAMP // NW // CHUNK
NB = 4

_mesh = plsc.VectorSubcoreMesh(
    core_axis_name="c", subcore_axis_name="s", num_cores=NC, num_subcores=NS)


@functools.partial(
    pl.kernel,
    out_type=(
        jax.ShapeDtypeStruct((BATCH, EMBED), jnp.float32),
        jax.ShapeDtypeStruct((BATCH, EMBED), jnp.float32),
        jax.ShapeDtypeStruct((BATCH * NSAMP, EMBED), jnp.float32),
    ),
    mesh=_mesh,
    scratch_types=(
        pltpu.VMEM((SMALL_CH, CHUNK), jnp.int32),
        pltpu.VMEM((SMALL_CH, CHUNK), jnp.int32),
        pltpu.VMEM((NOISE_CH, CHUNK), jnp.int32),
        pltpu.VMEM((NB, CHUNK, EMBED), jnp.float32),
        pltpu.SemaphoreType.DMA,
    ),
)
def _sc_gather(iw_h, ow_h, nz_h, tin_h, tout_h, o1_h, o2_h, o3_h,
               idxa_v, idxb_v, idxn_v, bufs_v, gsem):
    w = lax.axis_index("s") * NC + lax.axis_index("c")

    pltpu.sync_copy(iw_h.at[w], idxa_v)
    pltpu.sync_copy(ow_h.at[w], idxb_v)
    pltpu.sync_copy(nz_h.at[w], idxn_v)

    for idx_v, tab_h, out_h in ((idxa_v, tin_h, o1_h), (idxb_v, tout_h, o2_h)):
        descs = []
        for b in range(SMALL_CH):
            descs.append(
                pltpu.async_copy(tab_h.at[idx_v.at[b]], bufs_v.at[b], gsem))
        for b in range(SMALL_CH):
            descs[b].wait()
        for b in range(SMALL_CH):
            pltpu.sync_copy(
                bufs_v.at[b],
                out_h.at[pl.ds((w * SMALL_CH + b) * CHUNK, CHUNK)])

    base = w * (NOISE_CH * CHUNK)

    @pl.loop(0, NOISE_CH, step=NB)
    def _group(g):
        descs = []
        for b in range(NB):
            descs.append(
                pltpu.async_copy(tout_h.at[idxn_v.at[g + b]], bufs_v.at[b],
                                 gsem))
        for b in range(NB):
            descs[b].wait()
        for b in range(NB):
            pltpu.sync_copy(
                bufs_v.at[b],
                o3_h.at[pl.ds(base + (g + b) * CHUNK, CHUNK)])


def kernel(input_words, output_words, noise_words, in_embed, out_embed):
    iw = input_words.astype(jnp.int32).reshape(NW, SMALL_CH, CHUNK)
    ow = output_words.astype(jnp.int32).reshape(NW, SMALL_CH, CHUNK)
    nz = noise_words.astype(jnp.int32).reshape(NW, NOISE_CH, CHUNK)
    o1, o2, o3 = _sc_gather(iw, ow, nz, in_embed, out_embed)
    return o1, o2, o3.reshape(BATCH, NSAMP, EMBED)


# speedup vs baseline: 1.0166x; 1.0166x over previous
import functools
import jax
import jax.numpy as jnp
from jax import lax
from jax.experimental import pallas as pl
from jax.experimental.pallas import tpu as pltpu
from jax.experimental.pallas import tpu_sc as plsc

VOCAB = 100000
EMBED = 128
BATCH = 16384
NSAMP = 64

NC = 2
NS = 16
NW = NC * NS

CHUNK = 128
SMALL_CH = BATCH // NW // CHUNK
NOISE_CH = BATCH * NSAMP // NW // CHUNK
NB = 4

_mesh = plsc.VectorSubcoreMesh(
    core_axis_name="c", subcore_axis_name="s", num_cores=NC, num_subcores=NS)


@functools.partial(
    pl.kernel,
    out_type=(
        jax.ShapeDtypeStruct((BATCH, EMBED), jnp.float32),
        jax.ShapeDtypeStruct((BATCH, EMBED), jnp.float32),
        jax.ShapeDtypeStruct((BATCH * NSAMP, EMBED), jnp.float32),
    ),
    mesh=_mesh,
    scratch_types=(
        pltpu.VMEM((SMALL_CH, CHUNK), jnp.int32),
        pltpu.VMEM((SMALL_CH, CHUNK), jnp.int32),
        pltpu.VMEM((NOISE_CH, CHUNK), jnp.int32),
        pltpu.VMEM((NB, CHUNK, EMBED), jnp.float32),
        pltpu.SemaphoreType.DMA,
        pltpu.SemaphoreType.DMA,
    ),
)
def _sc_gather(iw_h, ow_h, nz_h, tin_h, tout_h, o1_h, o2_h, o3_h,
               idxa_v, idxb_v, idxn_v, bufs_v, gsemA, gsemB):
    w = lax.axis_index("s") * NC + lax.axis_index("c")

    pltpu.sync_copy(iw_h.at[w], idxa_v)
    pltpu.sync_copy(ow_h.at[w], idxb_v)
    pltpu.sync_copy(nz_h.at[w], idxn_v)

    for idx_v, tab_h, out_h in ((idxa_v, tin_h, o1_h), (idxb_v, tout_h, o2_h)):
        descs = []
        for b in range(SMALL_CH):
            sem = gsemA if b < 2 else gsemB
            descs.append(
                pltpu.async_copy(tab_h.at[idx_v.at[b]], bufs_v.at[b], sem))
        for pair in range(2):
            for b in (2 * pair, 2 * pair + 1):
                descs[b].wait()
            for b in (2 * pair, 2 * pair + 1):
                pltpu.sync_copy(
                    bufs_v.at[b],
                    out_h.at[pl.ds((w * SMALL_CH + b) * CHUNK, CHUNK)])

    base = w * (NOISE_CH * CHUNK)

    @pl.loop(0, NOISE_CH, step=NB)
    def _group(g):
        descs = []
        for b in range(NB):
            sem = gsemA if b < 2 else gsemB
            descs.append(
                pltpu.async_copy(tout_h.at[idxn_v.at[g + b]], bufs_v.at[b],
                                 sem))
        for pair in range(2):
            for b in (2 * pair, 2 * pair + 1):
                descs[b].wait()
            for b in (2 * pair, 2 * pair + 1):
                pltpu.sync_copy(
                    bufs_v.at[b],
                    o3_h.at[pl.ds(base + (g + b) * CHUNK, CHUNK)])


def kernel(input_words, output_words, noise_words, in_embed, out_embed):
    iw = input_words.astype(jnp.int32).reshape(NW, SMALL_CH, CHUNK)
    ow = output_words.astype(jnp.int32).reshape(NW, SMALL_CH, CHUNK)
    nz = noise_words.astype(jnp.int32).reshape(NW, NOISE_CH, CHUNK)
    o1, o2, o3 = _sc_gather(iw, ow, nz, in_embed, out_embed)
    return o1, o2, o3.reshape(BATCH, NSAMP, EMBED)


# speedup vs baseline: 1.0202x; 1.0035x over previous
import functools
import jax
import jax.numpy as jnp
from jax import lax
from jax.experimental import pallas as pl
from jax.experimental.pallas import tpu as pltpu
from jax.experimental.pallas import tpu_sc as plsc

VOCAB = 100000
EMBED = 128
BATCH = 16384
NSAMP = 64

NC = 2
NS = 16
NW = NC * NS

CHUNK = 128
SMALL_CH = BATCH // NW // CHUNK
NOISE_CH = BATCH * NSAMP // NW // CHUNK
NB = 4

_mesh = plsc.VectorSubcoreMesh(
    core_axis_name="c", subcore_axis_name="s", num_cores=NC, num_subcores=NS)


@functools.partial(
    pl.kernel,
    out_type=(
        jax.ShapeDtypeStruct((BATCH, EMBED), jnp.float32),
        jax.ShapeDtypeStruct((BATCH, EMBED), jnp.float32),
        jax.ShapeDtypeStruct((BATCH * NSAMP, EMBED), jnp.float32),
    ),
    mesh=_mesh,
    scratch_types=(
        pltpu.VMEM((SMALL_CH, CHUNK), jnp.int32),
        pltpu.VMEM((SMALL_CH, CHUNK), jnp.int32),
        pltpu.VMEM((NOISE_CH, CHUNK), jnp.int32),
        pltpu.VMEM((NB, CHUNK, EMBED), jnp.float32),
        pltpu.SemaphoreType.DMA,
        pltpu.SemaphoreType.DMA,
        pltpu.SemaphoreType.DMA,
        pltpu.SemaphoreType.DMA,
    ),
)
def _sc_gather(iw_h, ow_h, nz_h, tin_h, tout_h, o1_h, o2_h, o3_h,
               idxa_v, idxb_v, idxn_v, bufs_v, s0, s1, s2, s3):
    w = lax.axis_index("s") * NC + lax.axis_index("c")

    pltpu.sync_copy(iw_h.at[w], idxa_v)
    pltpu.sync_copy(ow_h.at[w], idxb_v)
    pltpu.sync_copy(nz_h.at[w], idxn_v)

    sems = (s0, s1, s2, s3)
    for idx_v, tab_h, out_h in ((idxa_v, tin_h, o1_h), (idxb_v, tout_h, o2_h)):
        descs = []
        for b in range(SMALL_CH):
            descs.append(
                pltpu.async_copy(tab_h.at[idx_v.at[b]], bufs_v.at[b], sems[b]))
        for b in range(SMALL_CH):
            descs[b].wait()
            pltpu.sync_copy(
                bufs_v.at[b],
                out_h.at[pl.ds((w * SMALL_CH + b) * CHUNK, CHUNK)])

    base = w * (NOISE_CH * CHUNK)

    @pl.loop(0, NOISE_CH, step=NB)
    def _group(g):
        descs = []
        for b in range(NB):
            descs.append(
                pltpu.async_copy(tout_h.at[idxn_v.at[g + b]], bufs_v.at[b],
                                 sems[b]))
        for b in range(NB):
            descs[b].wait()
            pltpu.sync_copy(
                bufs_v.at[b],
                o3_h.at[pl.ds(base + (g + b) * CHUNK, CHUNK)])


def kernel(input_words, output_words, noise_words, in_embed, out_embed):
    iw = input_words.astype(jnp.int32).reshape(NW, SMALL_CH, CHUNK)
    ow = output_words.astype(jnp.int32).reshape(NW, SMALL_CH, CHUNK)
    nz = noise_words.astype(jnp.int32).reshape(NW, NOISE_CH, CHUNK)
    o1, o2, o3 = _sc_gather(iw, ow, nz, in_embed, out_embed)
    return o1, o2, o3.reshape(BATCH, NSAMP, EMBED)


# speedup vs baseline: 1.1125x; 1.0905x over previous
import functools
import jax
import jax.numpy as jnp
from jax import lax
from jax.experimental import pallas as pl
from jax.experimental.pallas import tpu as pltpu
from jax.experimental.pallas import tpu_sc as plsc

VOCAB = 100000
EMBED = 128
BATCH = 16384
NSAMP = 64

NC = 2
NS = 16
NW = NC * NS

CHUNK = 128
SMALL_CH = BATCH // NW // CHUNK
NOISE_CH = BATCH * NSAMP // NW // CHUNK
NB = 4

_mesh = plsc.VectorSubcoreMesh(
    core_axis_name="c", subcore_axis_name="s", num_cores=NC, num_subcores=NS)


@functools.partial(
    pl.kernel,
    out_type=(
        jax.ShapeDtypeStruct((BATCH, EMBED), jnp.float32),
        jax.ShapeDtypeStruct((BATCH, EMBED), jnp.float32),
        jax.ShapeDtypeStruct((BATCH * NSAMP, EMBED), jnp.float32),
    ),
    mesh=_mesh,
    scratch_types=(
        pltpu.VMEM((SMALL_CH, CHUNK), jnp.int32),
        pltpu.VMEM((SMALL_CH, CHUNK), jnp.int32),
        pltpu.VMEM((NOISE_CH, CHUNK), jnp.int32),
        pltpu.VMEM((NB, CHUNK, EMBED), jnp.float32),
        pltpu.SemaphoreType.DMA,
    ),
)
def _sc_gather(iw_h, ow_h, nz_h, tin_h, tout_h, o1_h, o2_h, o3_h,
               idxa_v, idxb_v, idxn_v, bufs_v, gsem):
    w = lax.axis_index("s") * NC + lax.axis_index("c")

    pltpu.sync_copy(iw_h.at[w], idxa_v)
    pltpu.sync_copy(ow_h.at[w], idxb_v)
    pltpu.sync_copy(nz_h.at[w], idxn_v)

    for idx_v, tab_h, out_h in ((idxa_v, tin_h, o1_h), (idxb_v, tout_h, o2_h)):
        descs = []
        for b in range(SMALL_CH):
            descs.append(
                pltpu.async_copy(tab_h.at[idx_v.at[b]], bufs_v.at[b], gsem))
        for b in range(SMALL_CH):
            descs[b].wait()
            pltpu.sync_copy(
                bufs_v.at[b],
                out_h.at[pl.ds((w * SMALL_CH + b) * CHUNK, CHUNK)])

    base = w * (NOISE_CH * CHUNK)

    @pl.loop(0, NOISE_CH, step=NB)
    def _group(g):
        descs = []
        for b in range(NB):
            descs.append(
                pltpu.async_copy(tout_h.at[idxn_v.at[g + b]], bufs_v.at[b],
                                 gsem))
        for b in range(NB):
            descs[b].wait()
            pltpu.sync_copy(
                bufs_v.at[b],
                o3_h.at[pl.ds(base + (g + b) * CHUNK, CHUNK)])


def kernel(input_words, output_words, noise_words, in_embed, out_embed):
    iw = input_words.astype(jnp.int32).reshape(NW, SMALL_CH, CHUNK)
    ow = output_words.astype(jnp.int32).reshape(NW, SMALL_CH, CHUNK)
    nz = noise_words.astype(jnp.int32).reshape(NW, NOISE_CH, CHUNK)
    o1, o2, o3 = _sc_gather(iw, ow, nz, in_embed, out_embed)
    return o1, o2, o3.reshape(BATCH, NSAMP, EMBED)


# speedup vs baseline: 1.1140x; 1.0014x over previous
import functools
import jax
import jax.numpy as jnp
from jax import lax
from jax.experimental import pallas as pl
from jax.experimental.pallas import tpu as pltpu
from jax.experimental.pallas import tpu_sc as plsc

VOCAB = 100000
EMBED = 128
BATCH = 16384
NSAMP = 64

NC = 2
NS = 16
NW = NC * NS

CHUNK = 128
SMALL_CH = BATCH // NW // CHUNK
NOISE_CH = BATCH * NSAMP // NW // CHUNK
NB = 4

_mesh = plsc.VectorSubcoreMesh(
    core_axis_name="c", subcore_axis_name="s", num_cores=NC, num_subcores=NS)


@functools.partial(
    pl.kernel,
    out_type=(
        jax.ShapeDtypeStruct((BATCH, EMBED), jnp.float32),
        jax.ShapeDtypeStruct((BATCH, EMBED), jnp.float32),
        jax.ShapeDtypeStruct((BATCH * NSAMP, EMBED), jnp.float32),
    ),
    mesh=_mesh,
    scratch_types=(
        pltpu.VMEM((SMALL_CH, CHUNK), jnp.int32),
        pltpu.VMEM((SMALL_CH, CHUNK), jnp.int32),
        pltpu.VMEM((NOISE_CH, CHUNK), jnp.int32),
        pltpu.VMEM((NB, CHUNK, EMBED), jnp.float32),
        pltpu.SemaphoreType.DMA,
    ),
)
def _sc_gather(iw_h, ow_h, nz_h, tin_h, tout_h, o1_h, o2_h, o3_h,
               idxa_v, idxb_v, idxn_v, bufs_v, gsem):
    w = lax.axis_index("s") * NC + lax.axis_index("c")

    pltpu.sync_copy(iw_h.at[w], idxa_v)
    pltpu.sync_copy(ow_h.at[w], idxb_v)
    pltpu.sync_copy(nz_h.at[w], idxn_v)

    for idx_v, tab_h, out_h in ((idxa_v, tin_h, o1_h), (idxb_v, tout_h, o2_h)):
        descs = []
        for b in range(SMALL_CH):
            descs.append(
                pltpu.async_copy(tab_h.at[idx_v.at[b]], bufs_v.at[b], gsem))
        for b in range(SMALL_CH):
            descs[b].wait()
            pltpu.sync_copy(
                bufs_v.at[b],
                out_h.at[pl.ds((w * SMALL_CH + b) * CHUNK, CHUNK)])

    base = w * (NOISE_CH * CHUNK)

    @pl.loop(0, NOISE_CH, step=NB, unroll=2)
    def _group(g):
        descs = []
        for b in range(NB):
            descs.append(
                pltpu.async_copy(tout_h.at[idxn_v.at[g + b]], bufs_v.at[b],
                                 gsem))
        for b in range(NB):
            descs[b].wait()
            pltpu.sync_copy(
                bufs_v.at[b],
                o3_h.at[pl.ds(base + (g + b) * CHUNK, CHUNK)])


def kernel(input_words, output_words, noise_words, in_embed, out_embed):
    iw = input_words.astype(jnp.int32).reshape(NW, SMALL_CH, CHUNK)
    ow = output_words.astype(jnp.int32).reshape(NW, SMALL_CH, CHUNK)
    nz = noise_words.astype(jnp.int32).reshape(NW, NOISE_CH, CHUNK)
    o1, o2, o3 = _sc_gather(iw, ow, nz, in_embed, out_embed)
    return o1, o2, o3.reshape(BATCH, NSAMP, EMBED)
